# Initial kernel scaffold; baseline (speedup 1.0000x reference)
import jax
import jax.numpy as jnp
from jax.experimental import pallas as pl


def kernel(x, edge_index, edge_attr, W0, b0, W2, b2, W3, b3):
    raise NotImplementedError("write your pallas kernel here")


# speedup vs baseline: 5.1113x; 5.1113x over previous
import functools

import jax
import jax.numpy as jnp
from jax import lax
from jax.experimental import pallas as pl
from jax.experimental.pallas import tpu as pltpu
from jax.experimental.pallas import tpu_sc as plsc

N = 10000
E = 320000
H = 16
D_FEAT = 128
OUT = 128

_G = 128
_NGROUPS = E // _G
_INFO = plsc.get_sparse_core_info()
_NC = _INFO.num_cores
_NS = _INFO.num_subcores
_NW = _NC * _NS
_GPW = _NGROUPS // _NW
_EXTRA = _NGROUPS - _GPW * _NW
_RPT = N // _NS


def _sc_segment_sum(src2d, edge_attr, zeros_init):
  mesh = plsc.VectorSubcoreMesh(core_axis_name="c", subcore_axis_name="s")

  @functools.partial(
      pl.kernel,
      out_type=jax.ShapeDtypeStruct((_NC, N, H), jnp.float32),
      mesh=mesh,
      compiler_params=pltpu.CompilerParams(use_tc_tiling_on_sc=False),
      scratch_types=[
          pltpu.VMEM((_GPW, _G), jnp.int32),
          pltpu.VMEM((1, _G), jnp.int32),
          pltpu.VMEM((_G, H), jnp.float32),
          pltpu.VMEM((_G, H), jnp.float32),
          pltpu.VMEM_SHARED((N, H), jnp.float32),
          pltpu.SemaphoreType.DMA,
          pltpu.SemaphoreType.DMA,
      ],
  )
  def seg_sum(src_hbm, attr_hbm, zero_hbm, out_hbm,
              idx2d, exidx, rows0, rows1, acc, sem0, sem1):
    c = lax.axis_index("c")
    s = lax.axis_index("s")
    wid = s * _NC + c

    pltpu.sync_copy(zero_hbm.at[pl.ds(s * _RPT, _RPT)],
                    acc.at[pl.ds(s * _RPT, _RPT)])
    pltpu.sync_copy(src_hbm.at[pl.ds(wid * _GPW, _GPW)], idx2d)
    plsc.subcore_barrier()

    g0 = wid * _GPW
    bufs = ((rows0, sem0), (rows1, sem1))

    def issue(g_rel, buf, sem):
      pltpu.async_copy(attr_hbm.at[pl.ds((g0 + g_rel) * _G, _G)], buf, sem)

    def wait(buf, sem):
      pltpu.make_async_copy(attr_hbm.at[pl.ds(0, _G)], buf, sem).wait()

    issue(0, rows0, sem0)
    issue(1, rows1, sem1)

    def pair_body(p, carry):
      for b, (buf, sem) in enumerate(bufs):
        g_rel = 2 * p + b
        wait(buf, sem)
        pltpu.sync_copy(buf, acc.at[idx2d.at[g_rel]], add=True)

        @pl.when(g_rel + 2 < _GPW)
        def _():
          issue(g_rel + 2, buf, sem)
      return carry

    lax.fori_loop(0, _GPW // 2, pair_body, 0)

    @pl.when(wid < _EXTRA)
    def _():
      gx = _NW * _GPW + wid
      pltpu.sync_copy(src_hbm.at[pl.ds(gx, 1)], exidx)
      pltpu.sync_copy(attr_hbm.at[pl.ds(gx * _G, _G)], rows0)
      pltpu.sync_copy(rows0, acc.at[exidx.at[0]], add=True)

    plsc.subcore_barrier()
    pltpu.sync_copy(acc.at[pl.ds(s * _RPT, _RPT)],
                    out_hbm.at[c, pl.ds(s * _RPT, _RPT)])

  return seg_sum(src2d, edge_attr, zeros_init)


def _elu(t):
  return jnp.where(t > 0, t, jnp.exp(jnp.minimum(t, 0.0)) - 1.0)


def _mlp_body(x_ref, p_ref, w0a_ref, w0b_ref, b0_ref, w2_ref, b2_ref,
              w3_ref, b3_ref, o_ref):
  es = p_ref[0] + p_ref[1]
  t = (jnp.dot(x_ref[...], w0a_ref[...], preferred_element_type=jnp.float32)
       + jnp.dot(es, w0b_ref[...], preferred_element_type=jnp.float32)
       + b0_ref[...])
  h = _elu(t)
  h = _elu(jnp.dot(h, w2_ref[...], preferred_element_type=jnp.float32)
           + b2_ref[...])
  o_ref[...] = (jnp.dot(h, w3_ref[...], preferred_element_type=jnp.float32)
                + b3_ref[...])


def kernel(x, edge_index, edge_attr, W0, b0, W2, b2, W3, b3):
  src2d = edge_index[0].astype(jnp.int32).reshape(_NGROUPS, _G)
  zeros_init = jnp.zeros((N, H), jnp.float32)
  partials = _sc_segment_sum(src2d, edge_attr, zeros_init)

  out = pl.pallas_call(
      _mlp_body,
      out_shape=jax.ShapeDtypeStruct((N, OUT), jnp.float32),
  )(x, partials, W0[:D_FEAT], W0[D_FEAT:], b0.reshape(1, H),
    W2, b2.reshape(1, H), W3, b3.reshape(1, OUT))
  return out
